# Initial kernel scaffold; baseline (speedup 1.0000x reference)
import jax
import jax.numpy as jnp
from jax.experimental import pallas as pl


def kernel(n, m, logits):
    raise NotImplementedError("write your pallas kernel here")


# speedup vs baseline: 1.0214x; 1.0214x over previous
import functools

import jax
import jax.numpy as jnp
from jax import lax
from jax.experimental import pallas as pl
from jax.experimental.pallas import tpu as pltpu
from jax.experimental.pallas import tpu_sc as plsc

MAX_ATOMS = 1024
MAX_BONDS = 1024
BATCH = 16384

NW = 32
BPW = BATCH // NW
NG = BPW // 128


def _lse_body(x_ref, out_ref):
    x = x_ref[...]
    mx = jnp.max(x)
    s = jnp.sum(jnp.exp(x - mx))
    z = mx + jnp.log(s)
    out_ref[...] = jnp.full((8, 128), z, jnp.float32)


def _lse(logits):
    return pl.pallas_call(
        _lse_body,
        out_shape=jax.ShapeDtypeStruct((8, 128), jnp.float32),
    )(logits)


_mesh = plsc.VectorSubcoreMesh(core_axis_name="c", subcore_axis_name="s")


@functools.partial(
    pl.kernel,
    mesh=_mesh,
    out_type=jax.ShapeDtypeStruct((BATCH,), jnp.float32),
    scratch_types=[
        pltpu.VMEM((BPW,), jnp.int32),
        pltpu.VMEM((BPW,), jnp.int32),
        pltpu.VMEM((NG, 128), jnp.int32),
        pltpu.VMEM((BPW,), jnp.float32),
        pltpu.VMEM((16,), jnp.float32),
        pltpu.SemaphoreType.DMA,
    ],
)
def _sc_gather(flat_hbm, n_hbm, m_hbm, z_hbm, out_hbm,
               n_v, m_v, idx_v, val_v, z_v, sem):
    wid = lax.axis_index("s") * 2 + lax.axis_index("c")
    base = wid * BPW
    pltpu.sync_copy(n_hbm.at[pl.ds(base, BPW)], n_v)
    pltpu.sync_copy(m_hbm.at[pl.ds(base, BPW)], m_v)
    pltpu.sync_copy(z_hbm, z_v)
    for j in range(NG):
        for t in range(8):
            o = j * 128 + t * 16
            nn = n_v[pl.ds(o, 16)]
            mm = m_v[pl.ds(o, 16)]
            idx_v[j, pl.ds(t * 16, 16)] = nn * MAX_BONDS + mm
    copies = [
        pltpu.async_copy(flat_hbm.at[idx_v.at[j]],
                         val_v.at[pl.ds(j * 128, 128)], sem)
        for j in range(NG)
    ]
    for c in copies:
        c.wait()
    zz = z_v[...]
    for t in range(BPW // 16):
        o = t * 16
        val_v[pl.ds(o, 16)] = val_v[pl.ds(o, 16)] - zz
    pltpu.sync_copy(val_v, out_hbm.at[pl.ds(base, BPW)])


def kernel(n, m, logits):
    z8 = _lse(logits)
    zv = z8[0, :16]
    flat = logits.reshape(-1)
    return _sc_gather(flat, n.astype(jnp.int32), m.astype(jnp.int32), zv)
